# baseline (speedup 1.0000x reference)
import functools

import jax
import jax.numpy as jnp
from jax import lax
from jax.experimental import pallas as pl
from jax.experimental.pallas import tpu as pltpu
from jax.experimental.pallas import tpu_sc as plsc

B = 16384
U = 1000000
I = 1000000
H = 16
NC = 2
NS = 16
L = 16
NW = NC * NS
BPW = B // NW
CH = 128
NCH = BPW // CH
NB = BPW // L

_mesh = plsc.VectorSubcoreMesh(core_axis_name="c", subcore_axis_name="s",
                               num_cores=NC, num_subcores=NS)


@functools.partial(
    pl.kernel,
    out_type=(
        jax.ShapeDtypeStruct((B,), jnp.float32),
        jax.ShapeDtypeStruct((NW, L), jnp.float32),
    ),
    mesh=_mesh,
    compiler_params=pltpu.CompilerParams(needs_layout_passes=False,
                                         use_tc_tiling_on_sc=False),
    scratch_types=[
        pltpu.VMEM((NCH, CH), jnp.int32),
        pltpu.VMEM((NCH, CH), jnp.int32),
        pltpu.VMEM((BPW, H), jnp.float32),
        pltpu.VMEM((BPW, H), jnp.float32),
        pltpu.VMEM((BPW,), jnp.float32),
        pltpu.VMEM((BPW,), jnp.float32),
        pltpu.VMEM((BPW,), jnp.float32),
        pltpu.VMEM((BPW,), jnp.float32),
        pltpu.VMEM((L,), jnp.float32),
        pltpu.VMEM((L,), jnp.float32),
        pltpu.SemaphoreType.DMA,
    ],
)
def _mf_sc_kernel(user_h, item_h, rating_h, uw_h, iw_h, ub_h, ib_h, bias_h,
                  tgt_h, part_h,
                  uidx_v, iidx_v, urows_v, irows_v, ubr_v, ibr_v,
                  rat_v, out_v, sqa_v, bias_v, sem):
    wid = lax.axis_index("s") * NC + lax.axis_index("c")
    base = wid * BPW

    pltpu.sync_copy(user_h.at[pl.ds(wid * NCH, NCH)], uidx_v)
    pltpu.sync_copy(item_h.at[pl.ds(wid * NCH, NCH)], iidx_v)
    pltpu.sync_copy(rating_h.at[pl.ds(base, BPW)], rat_v)
    pltpu.sync_copy(bias_h, bias_v)

    copies = []
    for c in range(NCH):
        sl = pl.ds(c * CH, CH)
        copies.append(pltpu.async_copy(uw_h.at[uidx_v.at[c]], urows_v.at[sl], sem))
        copies.append(pltpu.async_copy(iw_h.at[iidx_v.at[c]], irows_v.at[sl], sem))
        copies.append(pltpu.async_copy(ub_h.at[uidx_v.at[c]], ubr_v.at[sl], sem))
        copies.append(pltpu.async_copy(ib_h.at[iidx_v.at[c]], ibr_v.at[sl], sem))
    for cp in copies:
        cp.wait()

    gbias = bias_v[...]
    lanes = lax.iota(jnp.int32, L)

    def block(b, sqacc):
        rows = b * L + lanes
        ub = ubr_v[pl.ds(b * L, L)]
        ib = ibr_v[pl.ds(b * L, L)]
        acc = jnp.zeros((L,), jnp.float32)
        for h in range(H):
            hcol = jnp.full((L,), h, jnp.int32)
            gu = plsc.load_gather(urows_v, [rows, hcol])
            gi = plsc.load_gather(irows_v, [rows, hcol])
            acc = acc + (gu + ub) * (gi + ib)
        out = acc + gbias
        out_v[pl.ds(b * L, L)] = out
        err = out - rat_v[pl.ds(b * L, L)]
        return sqacc + err * err

    sqacc = lax.fori_loop(0, NB, block, jnp.zeros((L,), jnp.float32))

    sqa_v[...] = sqacc
    pltpu.sync_copy(sqa_v, part_h.at[wid])
    pltpu.sync_copy(out_v, tgt_h.at[pl.ds(base, BPW)])


def kernel(user, item, rating, user_weight, item_weight, user_bias, item_bias,
           bias):
    user2d = user.astype(jnp.int32).reshape(NW * NCH, CH)
    item2d = item.astype(jnp.int32).reshape(NW * NCH, CH)
    bias16 = jnp.broadcast_to(bias.astype(jnp.float32), (L,))
    target, parts = _mf_sc_kernel(user2d, item2d, rating, user_weight,
                                  item_weight, user_bias.reshape(U),
                                  item_bias.reshape(I), bias16)
    loss = jnp.sum(parts) / B
    return (target, loss)
